# baseline (speedup 1.0000x reference)
import functools

import jax
import jax.numpy as jnp
from jax import lax
from jax.experimental import pallas as pl
from jax.experimental.pallas import tpu as pltpu
from jax.experimental.pallas import tpu_sc as plsc

N = 10000
E = 160000
D = 300
L = 5
DP = 384
NR = 160
NRANGE = 64
NPAD = NR * NRANGE
ACCR = 168
TRASH = 160
CHUNK = 64
CH = 48
CAPG = CH * CHUNK
BR = 400


def _sc_accumulate(row_src, init_src, gidx, didx, width, cadd):
    @functools.partial(
        pl.kernel,
        out_type=jax.ShapeDtypeStruct((NPAD, width), jnp.float32),
        mesh=plsc.VectorSubcoreMesh(core_axis_name="c", subcore_axis_name="s"),
        compiler_params=pltpu.CompilerParams(needs_layout_passes=False),
        scratch_types=[
            pltpu.VMEM((CH, CHUNK), jnp.int32),
            pltpu.VMEM((CH, CHUNK), jnp.int32),
            pltpu.VMEM((CHUNK, width), jnp.float32),
            pltpu.VMEM((CHUNK, width), jnp.float32),
            pltpu.VMEM((ACCR, width), jnp.float32),
            pltpu.SemaphoreType.DMA,
            pltpu.SemaphoreType.DMA,
        ],
    )
    def run(rows_hbm, init_hbm, gidx_hbm, didx_hbm, out_hbm,
            gidx_v, didx_v, buf0, buf1, acc_v, sem0, sem1):
        w = lax.axis_index("s") * 2 + lax.axis_index("c")
        iota16 = lax.iota(jnp.int32, 16)

        def add_chunk(k, buf):
            def group(gq, c2):
                e0 = gq * 16
                dvec = didx_v[k, pl.ds(e0, 16)]
                evec = e0 + iota16

                def colblk(cb, c3):
                    for j in range(16):
                        cvec = iota16 * 0 + (cb * 16 + j)
                        x = plsc.load_gather(buf, [evec, cvec])
                        plsc.addupdate_scatter(acc_v, [dvec, cvec], x)
                    return c3

                lax.fori_loop(0, cadd // 16, colblk, 0)
                return c2

            lax.fori_loop(0, CHUNK // 16, group, 0)

        for p in range(2):
            r = 2 * w + p
            rbase = r * NR
            pltpu.sync_copy(gidx_hbm.at[r], gidx_v)
            pltpu.sync_copy(didx_hbm.at[r], didx_v)
            pltpu.sync_copy(init_hbm.at[pl.ds(rbase, NR)],
                            acc_v.at[pl.ds(0, NR)])
            pltpu.async_copy(rows_hbm.at[gidx_v.at[0]], buf0, sem0)

            def pair(t, c1):
                k0 = 2 * t
                pltpu.make_async_copy(rows_hbm.at[gidx_v.at[k0]],
                                      buf0, sem0).wait()
                pltpu.async_copy(rows_hbm.at[gidx_v.at[k0 + 1]], buf1, sem1)
                add_chunk(k0, buf0)
                pltpu.make_async_copy(rows_hbm.at[gidx_v.at[k0 + 1]],
                                      buf1, sem1).wait()

                @pl.when(t + 1 < CH // 2)
                def _():
                    pltpu.async_copy(rows_hbm.at[gidx_v.at[k0 + 2]],
                                     buf0, sem0)

                add_chunk(k0 + 1, buf1)
                return c1

            lax.fori_loop(0, CH // 2, pair, 0)
            pltpu.sync_copy(acc_v.at[pl.ds(0, NR)],
                            out_hbm.at[pl.ds(rbase, NR)])

    return run(row_src, init_src, gidx, didx)


def _embed_body(x_ref, xe1_ref, xe2_ref, cnt_ref, combo_ref, sl_ref,
                h_ref, base_ref):
    lanes = lax.broadcasted_iota(jnp.int32, (1, 128), 1)
    oh0 = (x_ref[:, 0:1] == lanes).astype(jnp.float32)
    oh1 = (x_ref[:, 1:2] == lanes).astype(jnp.float32)
    h = (jnp.dot(oh0, xe1_ref[...], preferred_element_type=jnp.float32)
         + jnp.dot(oh1, xe2_ref[...], preferred_element_type=jnp.float32))
    h_ref[...] = h
    base_ref[...] = h + sl_ref[...] + jnp.dot(
        cnt_ref[...], combo_ref[...], preferred_element_type=jnp.float32)


def _embed_tc(x, xe1p, xe2p, cnt, combo0, sl0):
    return pl.pallas_call(
        _embed_body,
        grid=(N // BR,),
        in_specs=[
            pl.BlockSpec((BR, 2), lambda i: (i, 0)),
            pl.BlockSpec((128, DP), lambda i: (0, 0)),
            pl.BlockSpec((128, DP), lambda i: (0, 0)),
            pl.BlockSpec((BR, 128), lambda i: (i, 0)),
            pl.BlockSpec((128, DP), lambda i: (0, 0)),
            pl.BlockSpec((1, DP), lambda i: (0, 0)),
        ],
        out_specs=[pl.BlockSpec((BR, DP), lambda i: (i, 0))] * 2,
        out_shape=[jax.ShapeDtypeStruct((NPAD, DP), jnp.float32)] * 2,
    )(x, xe1p, xe2p, cnt, combo0, sl0)


def _mlp_body(a_ref, w1_ref, b1_ref, w2_ref, b2_ref, cnt_ref, combo_ref,
              sl_ref, h_ref, base_ref):
    z = jnp.maximum(
        jnp.dot(a_ref[...], w1_ref[...], preferred_element_type=jnp.float32)
        + b1_ref[...], 0.0)
    hm = (jnp.dot(z, w2_ref[...], preferred_element_type=jnp.float32)
          + b2_ref[...])
    h = jnp.maximum(hm, 0.0)
    h_ref[...] = h
    base_ref[...] = h + sl_ref[...] + jnp.dot(
        cnt_ref[...], combo_ref[...], preferred_element_type=jnp.float32)


def _mlp_tc(aggr, w1, b1, w2, b2, cnt, combo, sl):
    return pl.pallas_call(
        _mlp_body,
        grid=(N // BR,),
        in_specs=[
            pl.BlockSpec((BR, DP), lambda i: (i, 0)),
            pl.BlockSpec((DP, 2 * D), lambda i: (0, 0)),
            pl.BlockSpec((1, 2 * D), lambda i: (0, 0)),
            pl.BlockSpec((2 * D, DP), lambda i: (0, 0)),
            pl.BlockSpec((1, DP), lambda i: (0, 0)),
            pl.BlockSpec((BR, 128), lambda i: (i, 0)),
            pl.BlockSpec((128, DP), lambda i: (0, 0)),
            pl.BlockSpec((1, DP), lambda i: (0, 0)),
        ],
        out_specs=[pl.BlockSpec((BR, DP), lambda i: (i, 0))] * 2,
        out_shape=[jax.ShapeDtypeStruct((NPAD, DP), jnp.float32)] * 2,
    )(aggr, w1, b1, w2, b2, cnt, combo, sl)


def _mlp_last_body(a_ref, w1_ref, b1_ref, w2_ref, b2_ref, o_ref):
    z = jnp.maximum(
        jnp.dot(a_ref[...], w1_ref[...], preferred_element_type=jnp.float32)
        + b1_ref[...], 0.0)
    hm = (jnp.dot(z, w2_ref[...], preferred_element_type=jnp.float32)
          + b2_ref[...])
    o_ref[...] = hm[:, :D]


def _mlp_last_tc(aggr, w1, b1, w2, b2):
    return pl.pallas_call(
        _mlp_last_body,
        grid=(N // BR,),
        in_specs=[
            pl.BlockSpec((BR, DP), lambda i: (i, 0)),
            pl.BlockSpec((DP, 2 * D), lambda i: (0, 0)),
            pl.BlockSpec((1, 2 * D), lambda i: (0, 0)),
            pl.BlockSpec((2 * D, DP), lambda i: (0, 0)),
            pl.BlockSpec((1, DP), lambda i: (0, 0)),
        ],
        out_specs=pl.BlockSpec((BR, D), lambda i: (i, 0)),
        out_shape=jax.ShapeDtypeStruct((N, D), jnp.float32),
    )(aggr, w1, b1, w2, b2)


def _partition_edges(src, dst, combo):
    eid = jnp.arange(E, dtype=jnp.int32)
    g = dst // NR
    oh = g[:, None] == jnp.arange(NRANGE, dtype=g.dtype)[None, :]
    rk = (jnp.take_along_axis(jnp.cumsum(oh.astype(jnp.int32), axis=0),
                              g[:, None].astype(jnp.int32), axis=1)[:, 0] - 1)
    slot = jnp.where(rk < CAPG, g * CAPG + rk, NRANGE * CAPG)
    perm = jnp.full((NRANGE * CAPG,), E, jnp.int32).at[slot].set(eid,
                                                                 mode="drop")
    valid = perm < E
    pe = jnp.minimum(perm, E - 1)
    gslot = jnp.arange(NRANGE * CAPG, dtype=jnp.int32) // CAPG
    srcs = jnp.where(valid, src[pe], 0)
    dloc = jnp.where(valid, dst[pe] - gslot * NR, TRASH)
    qv = jnp.where(valid, combo[pe], 0)
    shape = (NRANGE, CH, CHUNK)
    return (srcs.reshape(shape).astype(jnp.int32),
            dloc.reshape(shape).astype(jnp.int32),
            qv.reshape(shape).astype(jnp.int32))


def kernel(x, edge_index, edge_attr, xe1, xe2, W1, b1, W2, b2,
           ee1, ee2, bn_g, bn_b, bn_m, bn_v):
    f32 = jnp.float32
    src = edge_index[0]
    dst = edge_index[1]
    combo = edge_attr[:, 0] * 3 + edge_attr[:, 1]
    sidx, didx, qidx = _partition_edges(src, dst, combo)

    scale = bn_g / jnp.sqrt(bn_v + 1e-5)
    W2f = W2 * scale[:, None, :]
    b2f = b2 * scale + bn_b - bn_m * scale
    W1p = jnp.zeros((L, DP, 2 * D), f32).at[:, :D, :].set(W1)
    W2p = jnp.zeros((L, 2 * D, DP), f32).at[:, :, :D].set(W2f)
    b2p = jnp.zeros((L, 1, DP), f32).at[:, 0, :D].set(b2f)
    b1p = b1.reshape(L, 1, 2 * D)
    xe1p = jnp.zeros((128, DP), f32).at[:120, :D].set(xe1)
    xe2p = jnp.zeros((128, DP), f32).at[:3, :D].set(xe2)
    r1 = jnp.repeat(jnp.arange(3), 3)
    r2 = jnp.tile(jnp.arange(3), 3)
    combop = jnp.zeros((L, 128, DP), f32).at[:, :9, :D].set(
        ee1[:, r1, :] + ee2[:, r2, :])
    slp = jnp.zeros((L, 1, DP), f32).at[:, 0, :D].set(
        ee1[:, 4, :] + ee2[:, 0, :])

    onehot16 = jnp.eye(16, 128, dtype=f32)
    zinit = jnp.zeros((NPAD, 128), f32)
    cnt = _sc_accumulate(onehot16, zinit, qidx, didx, 128, 16)

    h, base = _embed_tc(x, xe1p, xe2p, cnt, combop[0], slp[0])
    for l in range(L):
        aggr = _sc_accumulate(h, base, sidx, didx, DP, 304)
        if l < L - 1:
            h, base = _mlp_tc(aggr, W1p[l], b1p[l], W2p[l], b2p[l],
                              cnt, combop[l + 1], slp[l + 1])
        else:
            out = _mlp_last_tc(aggr, W1p[l], b1p[l], W2p[l], b2p[l])
    return out


# speedup vs baseline: 1.8257x; 1.8257x over previous
import functools

import jax
import jax.numpy as jnp
from jax import lax
from jax.experimental import pallas as pl
from jax.experimental.pallas import tpu as pltpu
from jax.experimental.pallas import tpu_sc as plsc

N = 10000
E = 160000
D = 300
L = 5
DP = 384
NR = 160
NRANGE = 64
NPAD = NR * NRANGE
ACCR = 168
TRASH = 160
CHUNK = 64
CH = 48
CAPG = CH * CHUNK
BR = 400


def _sc_accumulate(row_src, init_src, gidx, didx, width, cadd):
    @functools.partial(
        pl.kernel,
        out_type=jax.ShapeDtypeStruct((NPAD, width), jnp.float32),
        mesh=plsc.VectorSubcoreMesh(core_axis_name="c", subcore_axis_name="s"),
        compiler_params=pltpu.CompilerParams(needs_layout_passes=False),
        scratch_types=[
            pltpu.VMEM((CH, CHUNK), jnp.int32),
            pltpu.VMEM((CH, CHUNK), jnp.int32),
            pltpu.VMEM((CHUNK, width), jnp.float32),
            pltpu.VMEM((CHUNK, width), jnp.float32),
            pltpu.VMEM((ACCR, width), jnp.float32),
            pltpu.SemaphoreType.DMA,
            pltpu.SemaphoreType.DMA,
        ],
    )
    def run(rows_hbm, init_hbm, gidx_hbm, didx_hbm, out_hbm,
            gidx_v, didx_v, buf0, buf1, acc_v, sem0, sem1):
        w = lax.axis_index("s") * 2 + lax.axis_index("c")
        iota16 = lax.iota(jnp.int32, 16)

        def add_chunk(k, buf):
            @plsc.parallel_loop(0, CHUNK // 16, unroll=2)
            def group(gq):
                e0 = gq * 16
                dvec = didx_v[k, pl.ds(e0, 16)]
                for i in range(16):
                    d = dvec[i]
                    row = [buf[e0 + i, pl.ds(j * 16, 16)]
                           for j in range(cadd // 16)]
                    for j in range(cadd // 16):
                        plsc.addupdate(acc_v.at[d, pl.ds(j * 16, 16)], row[j])

        for p in range(2):
            r = 2 * w + p
            rbase = r * NR
            pltpu.sync_copy(gidx_hbm.at[r], gidx_v)
            pltpu.sync_copy(didx_hbm.at[r], didx_v)
            pltpu.sync_copy(init_hbm.at[pl.ds(rbase, NR)],
                            acc_v.at[pl.ds(0, NR)])
            pltpu.async_copy(rows_hbm.at[gidx_v.at[0]], buf0, sem0)

            def pair(t, c1):
                k0 = 2 * t
                pltpu.make_async_copy(rows_hbm.at[gidx_v.at[k0]],
                                      buf0, sem0).wait()
                pltpu.async_copy(rows_hbm.at[gidx_v.at[k0 + 1]], buf1, sem1)
                add_chunk(k0, buf0)
                pltpu.make_async_copy(rows_hbm.at[gidx_v.at[k0 + 1]],
                                      buf1, sem1).wait()

                @pl.when(t + 1 < CH // 2)
                def _():
                    pltpu.async_copy(rows_hbm.at[gidx_v.at[k0 + 2]],
                                     buf0, sem0)

                add_chunk(k0 + 1, buf1)
                return c1

            lax.fori_loop(0, CH // 2, pair, 0)
            pltpu.sync_copy(acc_v.at[pl.ds(0, NR)],
                            out_hbm.at[pl.ds(rbase, NR)])

    return run(row_src, init_src, gidx, didx)


def _embed_body(x_ref, xe1_ref, xe2_ref, cnt_ref, combo_ref, sl_ref,
                h_ref, base_ref):
    lanes = lax.broadcasted_iota(jnp.int32, (1, 128), 1)
    oh0 = (x_ref[:, 0:1] == lanes).astype(jnp.float32)
    oh1 = (x_ref[:, 1:2] == lanes).astype(jnp.float32)
    h = (jnp.dot(oh0, xe1_ref[...], preferred_element_type=jnp.float32)
         + jnp.dot(oh1, xe2_ref[...], preferred_element_type=jnp.float32))
    h_ref[...] = h
    base_ref[...] = h + sl_ref[...] + jnp.dot(
        cnt_ref[...], combo_ref[...], preferred_element_type=jnp.float32)


def _embed_tc(x, xe1p, xe2p, cnt, combo0, sl0):
    return pl.pallas_call(
        _embed_body,
        grid=(N // BR,),
        in_specs=[
            pl.BlockSpec((BR, 2), lambda i: (i, 0)),
            pl.BlockSpec((128, DP), lambda i: (0, 0)),
            pl.BlockSpec((128, DP), lambda i: (0, 0)),
            pl.BlockSpec((BR, 128), lambda i: (i, 0)),
            pl.BlockSpec((128, DP), lambda i: (0, 0)),
            pl.BlockSpec((1, DP), lambda i: (0, 0)),
        ],
        out_specs=[pl.BlockSpec((BR, DP), lambda i: (i, 0))] * 2,
        out_shape=[jax.ShapeDtypeStruct((NPAD, DP), jnp.float32)] * 2,
    )(x, xe1p, xe2p, cnt, combo0, sl0)


def _mlp_body(a_ref, w1_ref, b1_ref, w2_ref, b2_ref, cnt_ref, combo_ref,
              sl_ref, h_ref, base_ref):
    z = jnp.maximum(
        jnp.dot(a_ref[...], w1_ref[...], preferred_element_type=jnp.float32)
        + b1_ref[...], 0.0)
    hm = (jnp.dot(z, w2_ref[...], preferred_element_type=jnp.float32)
          + b2_ref[...])
    h = jnp.maximum(hm, 0.0)
    h_ref[...] = h
    base_ref[...] = h + sl_ref[...] + jnp.dot(
        cnt_ref[...], combo_ref[...], preferred_element_type=jnp.float32)


def _mlp_tc(aggr, w1, b1, w2, b2, cnt, combo, sl):
    return pl.pallas_call(
        _mlp_body,
        grid=(N // BR,),
        in_specs=[
            pl.BlockSpec((BR, DP), lambda i: (i, 0)),
            pl.BlockSpec((DP, 2 * D), lambda i: (0, 0)),
            pl.BlockSpec((1, 2 * D), lambda i: (0, 0)),
            pl.BlockSpec((2 * D, DP), lambda i: (0, 0)),
            pl.BlockSpec((1, DP), lambda i: (0, 0)),
            pl.BlockSpec((BR, 128), lambda i: (i, 0)),
            pl.BlockSpec((128, DP), lambda i: (0, 0)),
            pl.BlockSpec((1, DP), lambda i: (0, 0)),
        ],
        out_specs=[pl.BlockSpec((BR, DP), lambda i: (i, 0))] * 2,
        out_shape=[jax.ShapeDtypeStruct((NPAD, DP), jnp.float32)] * 2,
    )(aggr, w1, b1, w2, b2, cnt, combo, sl)


def _mlp_last_body(a_ref, w1_ref, b1_ref, w2_ref, b2_ref, o_ref):
    z = jnp.maximum(
        jnp.dot(a_ref[...], w1_ref[...], preferred_element_type=jnp.float32)
        + b1_ref[...], 0.0)
    hm = (jnp.dot(z, w2_ref[...], preferred_element_type=jnp.float32)
          + b2_ref[...])
    o_ref[...] = hm[:, :D]


def _mlp_last_tc(aggr, w1, b1, w2, b2):
    return pl.pallas_call(
        _mlp_last_body,
        grid=(N // BR,),
        in_specs=[
            pl.BlockSpec((BR, DP), lambda i: (i, 0)),
            pl.BlockSpec((DP, 2 * D), lambda i: (0, 0)),
            pl.BlockSpec((1, 2 * D), lambda i: (0, 0)),
            pl.BlockSpec((2 * D, DP), lambda i: (0, 0)),
            pl.BlockSpec((1, DP), lambda i: (0, 0)),
        ],
        out_specs=pl.BlockSpec((BR, D), lambda i: (i, 0)),
        out_shape=jax.ShapeDtypeStruct((N, D), jnp.float32),
    )(aggr, w1, b1, w2, b2)


def _partition_edges(src, dst, combo):
    eid = jnp.arange(E, dtype=jnp.int32)
    g = dst // NR
    oh = g[:, None] == jnp.arange(NRANGE, dtype=g.dtype)[None, :]
    rk = (jnp.take_along_axis(jnp.cumsum(oh.astype(jnp.int32), axis=0),
                              g[:, None].astype(jnp.int32), axis=1)[:, 0] - 1)
    slot = jnp.where(rk < CAPG, g * CAPG + rk, NRANGE * CAPG)
    perm = jnp.full((NRANGE * CAPG,), E, jnp.int32).at[slot].set(eid,
                                                                 mode="drop")
    valid = perm < E
    pe = jnp.minimum(perm, E - 1)
    gslot = jnp.arange(NRANGE * CAPG, dtype=jnp.int32) // CAPG
    srcs = jnp.where(valid, src[pe], 0)
    dloc = jnp.where(valid, dst[pe] - gslot * NR, TRASH)
    qv = jnp.where(valid, combo[pe], 0)
    shape = (NRANGE, CH, CHUNK)
    return (srcs.reshape(shape).astype(jnp.int32),
            dloc.reshape(shape).astype(jnp.int32),
            qv.reshape(shape).astype(jnp.int32))


def kernel(x, edge_index, edge_attr, xe1, xe2, W1, b1, W2, b2,
           ee1, ee2, bn_g, bn_b, bn_m, bn_v):
    f32 = jnp.float32
    src = edge_index[0]
    dst = edge_index[1]
    combo = edge_attr[:, 0] * 3 + edge_attr[:, 1]
    sidx, didx, qidx = _partition_edges(src, dst, combo)

    scale = bn_g / jnp.sqrt(bn_v + 1e-5)
    W2f = W2 * scale[:, None, :]
    b2f = b2 * scale + bn_b - bn_m * scale
    W1p = jnp.zeros((L, DP, 2 * D), f32).at[:, :D, :].set(W1)
    W2p = jnp.zeros((L, 2 * D, DP), f32).at[:, :, :D].set(W2f)
    b2p = jnp.zeros((L, 1, DP), f32).at[:, 0, :D].set(b2f)
    b1p = b1.reshape(L, 1, 2 * D)
    xe1p = jnp.zeros((128, DP), f32).at[:120, :D].set(xe1)
    xe2p = jnp.zeros((128, DP), f32).at[:3, :D].set(xe2)
    r1 = jnp.repeat(jnp.arange(3), 3)
    r2 = jnp.tile(jnp.arange(3), 3)
    combop = jnp.zeros((L, 128, DP), f32).at[:, :9, :D].set(
        ee1[:, r1, :] + ee2[:, r2, :])
    slp = jnp.zeros((L, 1, DP), f32).at[:, 0, :D].set(
        ee1[:, 4, :] + ee2[:, 0, :])

    onehot16 = jnp.eye(16, 128, dtype=f32)
    zinit = jnp.zeros((NPAD, 128), f32)
    cnt = _sc_accumulate(onehot16, zinit, qidx, didx, 128, 16)

    h, base = _embed_tc(x, xe1p, xe2p, cnt, combop[0], slp[0])
    for l in range(L):
        aggr = _sc_accumulate(h, base, sidx, didx, DP, 304)
        if l < L - 1:
            h, base = _mlp_tc(aggr, W1p[l], b1p[l], W2p[l], b2p[l],
                              cnt, combop[l + 1], slp[l + 1])
        else:
            out = _mlp_last_tc(aggr, W1p[l], b1p[l], W2p[l], b2p[l])
    return out
